# Initial kernel scaffold; baseline (speedup 1.0000x reference)
import jax
import jax.numpy as jnp
from jax.experimental import pallas as pl


def kernel(x, adj_t, W1, b1, W2, b2):
    raise NotImplementedError("write your pallas kernel here")


# speedup vs baseline: 4.7788x; 4.7788x over previous
import functools

import jax
import jax.numpy as jnp
from jax import lax
from jax.experimental import pallas as pl
from jax.experimental.pallas import tpu as pltpu
from jax.experimental.pallas import tpu_sc as plsc

N = 10000
E = 320000
D_IN = 128
D_HID = 256
D_OUT = 128

NC = 2
NT = 16
K = 128
EP = 327680
EC = EP // K
NH = N // NC
TRASH = 256
NACC = NH + TRASH + 8
NPADDEG = 10240
BN = 1000
BE = EC // (N // BN)

_f32 = jnp.float32


def _mesh():
    return plsc.VectorSubcoreMesh(core_axis_name="c", subcore_axis_name="s")


def _tile_rows(s, total, fn):
    rt8 = (total // (8 * NT)) * 8
    last = total - (NT - 1) * rt8

    @pl.when(s < NT - 1)
    def _():
        fn(s * rt8, rt8)

    @pl.when(s == NT - 1)
    def _():
        fn((NT - 1) * rt8, last)


@functools.partial(
    pl.kernel,
    out_type=[
        jax.ShapeDtypeStruct((NPADDEG,), _f32),
        jax.ShapeDtypeStruct((NPADDEG,), _f32),
    ],
    mesh=_mesh(),
    scratch_types=[
        pltpu.VMEM((EC // (NC * NT), K), jnp.int32),
        pltpu.VMEM((K,), _f32),
        pltpu.VMEM((NPADDEG // NT,), _f32),
        pltpu.VMEM_SHARED((NPADDEG,), _f32),
    ],
)
def _sc_deg(dst2d, dega, degb, dst_v, ones_v, zbuf, acc):
    c = lax.axis_index("c")
    s = lax.axis_index("s")
    nch = EC // (NC * NT)
    zchunk = NPADDEG // NT
    base = c * (EC // NC) + s * nch
    pltpu.sync_copy(dst2d.at[pl.ds(base, nch)], dst_v)
    for i in range(zchunk // 16):
        zbuf[pl.ds(16 * i, 16)] = jnp.zeros((16,), _f32)
    for i in range(K // 16):
        ones_v[pl.ds(16 * i, 16)] = jnp.ones((16,), _f32)
    pltpu.sync_copy(zbuf, acc.at[pl.ds(s * zchunk, zchunk)])
    plsc.subcore_barrier()

    def body(j, carry):
        pltpu.sync_copy(ones_v, acc.at[dst_v.at[j]], add=True)
        return carry

    lax.fori_loop(0, nch, body, 0)
    plsc.subcore_barrier()

    @pl.when(c == 0)
    def _():
        pltpu.sync_copy(acc.at[pl.ds(s * zchunk, zchunk)],
                        dega.at[pl.ds(s * zchunk, zchunk)])

    @pl.when(c == 1)
    def _():
        pltpu.sync_copy(acc.at[pl.ds(s * zchunk, zchunk)],
                        degb.at[pl.ds(s * zchunk, zchunk)])


@functools.partial(
    pl.kernel,
    out_type=jax.ShapeDtypeStruct((N, 128), _f32),
    mesh=_mesh(),
    scratch_types=[
        pltpu.VMEM((EC // NT, K), jnp.int32),
        pltpu.VMEM((EC // NT, K), jnp.int32),
        pltpu.VMEM((K, 128), _f32),
        pltpu.VMEM((K, 128), _f32),
        pltpu.VMEM_SHARED((NACC, 128), _f32),
        pltpu.SemaphoreType.DMA,
        pltpu.SemaphoreType.DMA,
    ],
)
def _sc_agg(src2d, dst0_2d, dst1_2d, hs, dum, out,
            src_v, dst_v, buf0, buf1, acc, sem0, sem1):
    c = lax.axis_index("c")
    s = lax.axis_index("s")
    nch = EC // NT
    pltpu.sync_copy(src2d.at[pl.ds(s * nch, nch)], src_v)

    @pl.when(c == 0)
    def _():
        pltpu.sync_copy(dst0_2d.at[pl.ds(s * nch, nch)], dst_v)

    @pl.when(c == 1)
    def _():
        pltpu.sync_copy(dst1_2d.at[pl.ds(s * nch, nch)], dst_v)

    def seed(off, size):
        pltpu.sync_copy(hs.at[pl.ds(c * NH + off, size)],
                        acc.at[pl.ds(off, size)])

    _tile_rows(s, NH, seed)
    plsc.subcore_barrier()

    pltpu.async_copy(hs.at[src_v.at[0]], buf0, sem0)

    def body(i, carry):
        j = 2 * i
        pltpu.async_copy(hs.at[src_v.at[j + 1]], buf1, sem1)
        pltpu.make_async_copy(dum, buf0, sem0).wait()
        pltpu.sync_copy(buf0, acc.at[dst_v.at[j]], add=True)

        @pl.when(j + 2 < nch)
        def _():
            pltpu.async_copy(hs.at[src_v.at[j + 2]], buf0, sem0)

        pltpu.make_async_copy(dum, buf1, sem1).wait()
        pltpu.sync_copy(buf1, acc.at[dst_v.at[j + 1]], add=True)
        return carry

    lax.fori_loop(0, nch // 2, body, 0)
    plsc.subcore_barrier()

    def writeout(off, size):
        pltpu.sync_copy(acc.at[pl.ds(off, size)],
                        out.at[pl.ds(c * NH + off, size)])

    _tile_rows(s, NH, writeout)


def _dst_transform(d, core):
    t = d - core * NH
    invalid = (t < 0) | (t >= NH)
    trash = NH + jnp.bitwise_and(d, TRASH - 1)
    return jnp.where(invalid, trash, t)


def _tc1_body(x_ref, w1_ref, dpa_ref, dpb_ref, dst_ref,
              hsa_ref, hsb_ref, dinv_ref, dst0_ref, dst1_ref):
    dinv = lax.rsqrt(dpa_ref[...] + dpb_ref[...] + 1.0)
    h = jnp.dot(x_ref[...], w1_ref[...], preferred_element_type=_f32)
    hs = h * dinv
    hsa_ref[...] = hs[:, : D_HID // 2]
    hsb_ref[...] = hs[:, D_HID // 2:]
    dinv_ref[...] = dinv
    d = dst_ref[...]
    dst0_ref[...] = _dst_transform(d, 0)
    dst1_ref[...] = _dst_transform(d, 1)


def _tc1(x, W1, dpa, dpb, dst2d):
    grid = (N // BN,)
    return pl.pallas_call(
        _tc1_body,
        grid=grid,
        in_specs=[
            pl.BlockSpec((BN, D_IN), lambda i: (i, 0)),
            pl.BlockSpec((D_IN, D_HID), lambda i: (0, 0)),
            pl.BlockSpec((BN, 1), lambda i: (i, 0)),
            pl.BlockSpec((BN, 1), lambda i: (i, 0)),
            pl.BlockSpec((BE, K), lambda i: (i, 0)),
        ],
        out_specs=[
            pl.BlockSpec((BN, D_HID // 2), lambda i: (i, 0)),
            pl.BlockSpec((BN, D_HID // 2), lambda i: (i, 0)),
            pl.BlockSpec((BN, 1), lambda i: (i, 0)),
            pl.BlockSpec((BE, K), lambda i: (i, 0)),
            pl.BlockSpec((BE, K), lambda i: (i, 0)),
        ],
        out_shape=[
            jax.ShapeDtypeStruct((N, D_HID // 2), _f32),
            jax.ShapeDtypeStruct((N, D_HID // 2), _f32),
            jax.ShapeDtypeStruct((N, 1), _f32),
            jax.ShapeDtypeStruct((EC, K), jnp.int32),
            jax.ShapeDtypeStruct((EC, K), jnp.int32),
        ],
    )(x, W1, dpa, dpb, dst2d)


def _tc2_body(agga_ref, aggb_ref, dinv_ref, b1a_ref, b1b_ref,
              w2a_ref, w2b_ref, hs2_ref):
    dinv = dinv_ref[...]
    ta = jax.nn.relu(agga_ref[...] * dinv + b1a_ref[...])
    tb = jax.nn.relu(aggb_ref[...] * dinv + b1b_ref[...])
    h2 = (jnp.dot(ta, w2a_ref[...], preferred_element_type=_f32)
          + jnp.dot(tb, w2b_ref[...], preferred_element_type=_f32))
    hs2_ref[...] = h2 * dinv


def _tc2(agg_a, agg_b, dinv, b1a, b1b, W2a, W2b):
    grid = (N // BN,)
    half = D_HID // 2
    return pl.pallas_call(
        _tc2_body,
        grid=grid,
        in_specs=[
            pl.BlockSpec((BN, half), lambda i: (i, 0)),
            pl.BlockSpec((BN, half), lambda i: (i, 0)),
            pl.BlockSpec((BN, 1), lambda i: (i, 0)),
            pl.BlockSpec((1, half), lambda i: (0, 0)),
            pl.BlockSpec((1, half), lambda i: (0, 0)),
            pl.BlockSpec((half, D_OUT), lambda i: (0, 0)),
            pl.BlockSpec((half, D_OUT), lambda i: (0, 0)),
        ],
        out_specs=pl.BlockSpec((BN, D_OUT), lambda i: (i, 0)),
        out_shape=jax.ShapeDtypeStruct((N, D_OUT), _f32),
    )(agg_a, agg_b, dinv, b1a, b1b, W2a, W2b)


def _tc3_body(p_ref, dinv_ref, b2_ref, out_ref):
    out_ref[...] = p_ref[...] * dinv_ref[...] + b2_ref[...]


def _tc3(p, dinv, b2r):
    grid = (N // BN,)
    return pl.pallas_call(
        _tc3_body,
        grid=grid,
        in_specs=[
            pl.BlockSpec((BN, D_OUT), lambda i: (i, 0)),
            pl.BlockSpec((BN, 1), lambda i: (i, 0)),
            pl.BlockSpec((1, D_OUT), lambda i: (0, 0)),
        ],
        out_specs=pl.BlockSpec((BN, D_OUT), lambda i: (i, 0)),
        out_shape=jax.ShapeDtypeStruct((N, D_OUT), _f32),
    )(p, dinv, b2r)


def kernel(x, adj_t, W1, b1, W2, b2):
    assert x.shape == (N, D_IN) and adj_t.shape == (2, E)
    assert W1.shape == (D_IN, D_HID) and W2.shape == (D_HID, D_OUT)

    pad = EP - E
    src2d = jnp.concatenate(
        [adj_t[0], jnp.zeros((pad,), jnp.int32)]).reshape(EC, K)
    dst_p = jnp.concatenate(
        [adj_t[1], jnp.full((pad,), N, jnp.int32)]).reshape(EC, K)
    dum = jnp.zeros((K, 128), _f32)

    dega, degb = _sc_deg(dst_p)
    dpa = dega[:N].reshape(N, 1)
    dpb = degb[:N].reshape(N, 1)

    hs_a, hs_b, dinv, dst0, dst1 = _tc1(x, W1, dpa, dpb, dst_p)
    agg_a = _sc_agg(src2d, dst0, dst1, hs_a, dum)
    agg_b = _sc_agg(src2d, dst0, dst1, hs_b, dum)

    b1a = b1[: D_HID // 2].reshape(1, -1)
    b1b = b1[D_HID // 2:].reshape(1, -1)
    hs2 = _tc2(agg_a, agg_b, dinv, b1a, b1b, W2[: D_HID // 2], W2[D_HID // 2:])

    p = _sc_agg(src2d, dst0, dst1, hs2, dum)
    return _tc3(p, dinv, b2.reshape(1, -1))


# speedup vs baseline: 4.8895x; 1.0232x over previous
import functools

import jax
import jax.numpy as jnp
from jax import lax
from jax.experimental import pallas as pl
from jax.experimental.pallas import tpu as pltpu
from jax.experimental.pallas import tpu_sc as plsc

N = 10000
E = 320000
D_IN = 128
D_HID = 256
D_OUT = 128

NC = 2
NT = 16
K = 128
EP = 327680
EC = EP // K
NH = N // NC
TRASH = 64
NACC = NH + TRASH + 8
NPADDEG = 10240
BN = 1000
BE = EC // (N // BN)

_f32 = jnp.float32


def _mesh():
    return plsc.VectorSubcoreMesh(core_axis_name="c", subcore_axis_name="s")


def _tile_rows(s, total, fn):
    rt8 = (total // (8 * NT)) * 8
    last = total - (NT - 1) * rt8

    @pl.when(s < NT - 1)
    def _():
        fn(s * rt8, rt8)

    @pl.when(s == NT - 1)
    def _():
        fn((NT - 1) * rt8, last)


@functools.partial(
    pl.kernel,
    out_type=[
        jax.ShapeDtypeStruct((NPADDEG,), _f32),
        jax.ShapeDtypeStruct((NPADDEG,), _f32),
    ],
    mesh=_mesh(),
    scratch_types=[
        pltpu.VMEM((EC // (NC * NT), K), jnp.int32),
        pltpu.VMEM((K,), _f32),
        pltpu.VMEM((NPADDEG // NT,), _f32),
        pltpu.VMEM_SHARED((NPADDEG,), _f32),
    ],
)
def _sc_deg(dst2d, dega, degb, dst_v, ones_v, zbuf, acc):
    c = lax.axis_index("c")
    s = lax.axis_index("s")
    nch = EC // (NC * NT)
    zchunk = NPADDEG // NT
    base = c * (EC // NC) + s * nch
    pltpu.sync_copy(dst2d.at[pl.ds(base, nch)], dst_v)
    for i in range(zchunk // 16):
        zbuf[pl.ds(16 * i, 16)] = jnp.zeros((16,), _f32)
    for i in range(K // 16):
        ones_v[pl.ds(16 * i, 16)] = jnp.ones((16,), _f32)
    pltpu.sync_copy(zbuf, acc.at[pl.ds(s * zchunk, zchunk)])
    plsc.subcore_barrier()

    def body(j, carry):
        pltpu.sync_copy(ones_v, acc.at[dst_v.at[j]], add=True)
        return carry

    lax.fori_loop(0, nch, body, 0)
    plsc.subcore_barrier()

    @pl.when(c == 0)
    def _():
        pltpu.sync_copy(acc.at[pl.ds(s * zchunk, zchunk)],
                        dega.at[pl.ds(s * zchunk, zchunk)])

    @pl.when(c == 1)
    def _():
        pltpu.sync_copy(acc.at[pl.ds(s * zchunk, zchunk)],
                        degb.at[pl.ds(s * zchunk, zchunk)])


def _agg_half(c, s, nch, hs, dum, out, src_v, dst_v, bufs, gsems, acc):
    def seed(off, size):
        pltpu.sync_copy(hs.at[pl.ds(c * NH + off, size)],
                        acc.at[pl.ds(off, size)])

    _tile_rows(s, NH, seed)
    plsc.subcore_barrier()

    def gather(j, q):
        pltpu.async_copy(hs.at[src_v.at[j]], bufs[q], gsems[q])

    def wait_gather(q):
        pltpu.make_async_copy(dum, bufs[q], gsems[q]).wait()

    gather(0, 0)
    gather(1, 1)

    def body(i, carry):
        b = 3 * i
        for q in range(3):
            p = b + q
            q2 = (q + 2) % 3

            @pl.when(p + 2 < nch)
            def _():
                gather(p + 2, q2)

            @pl.when(p < nch)
            def _():
                wait_gather(q)
                pltpu.sync_copy(bufs[q], acc.at[dst_v.at[p]], add=True)
        return carry

    lax.fori_loop(0, (nch + 2) // 3, body, 0)
    plsc.subcore_barrier()

    def writeout(off, size):
        pltpu.sync_copy(acc.at[pl.ds(off, size)],
                        out.at[pl.ds(c * NH + off, size)])

    _tile_rows(s, NH, writeout)


def _stage_indices(c, s, nch, src2d, dst0_2d, dst1_2d, src_v, dst_v):
    pltpu.sync_copy(src2d.at[pl.ds(s * nch, nch)], src_v)

    @pl.when(c == 0)
    def _():
        pltpu.sync_copy(dst0_2d.at[pl.ds(s * nch, nch)], dst_v)

    @pl.when(c == 1)
    def _():
        pltpu.sync_copy(dst1_2d.at[pl.ds(s * nch, nch)], dst_v)


_AGG_SCRATCH = [
    pltpu.VMEM((EC // NT, K), jnp.int32),
    pltpu.VMEM((EC // NT, K), jnp.int32),
    pltpu.VMEM((K, 128), _f32),
    pltpu.VMEM((K, 128), _f32),
    pltpu.VMEM((K, 128), _f32),
    pltpu.VMEM_SHARED((NACC, 128), _f32),
    pltpu.SemaphoreType.DMA,
    pltpu.SemaphoreType.DMA,
    pltpu.SemaphoreType.DMA,
]


@functools.partial(
    pl.kernel,
    out_type=[jax.ShapeDtypeStruct((N, 128), _f32)] * 2,
    mesh=_mesh(),
    scratch_types=_AGG_SCRATCH,
)
def _sc_agg_dual(src2d, dst0_2d, dst1_2d, hs_a, hs_b, dum, out_a, out_b,
                 src_v, dst_v, b0, b1, b2, acc, g0, g1, g2):
    c = lax.axis_index("c")
    s = lax.axis_index("s")
    nch = EC // NT
    _stage_indices(c, s, nch, src2d, dst0_2d, dst1_2d, src_v, dst_v)
    _agg_half(c, s, nch, hs_a, dum, out_a, src_v, dst_v, (b0, b1, b2),
              (g0, g1, g2), acc)
    plsc.subcore_barrier()
    _agg_half(c, s, nch, hs_b, dum, out_b, src_v, dst_v, (b0, b1, b2),
              (g0, g1, g2), acc)


@functools.partial(
    pl.kernel,
    out_type=jax.ShapeDtypeStruct((N, 128), _f32),
    mesh=_mesh(),
    scratch_types=_AGG_SCRATCH,
)
def _sc_agg(src2d, dst0_2d, dst1_2d, hs, dum, out,
            src_v, dst_v, b0, b1, b2, acc, g0, g1, g2):
    c = lax.axis_index("c")
    s = lax.axis_index("s")
    nch = EC // NT
    _stage_indices(c, s, nch, src2d, dst0_2d, dst1_2d, src_v, dst_v)
    _agg_half(c, s, nch, hs, dum, out, src_v, dst_v, (b0, b1, b2),
              (g0, g1, g2), acc)


def _dst_transform(d, core):
    t = d - core * NH
    invalid = (t < 0) | (t >= NH)
    trash = NH + jnp.bitwise_and(d, TRASH - 1)
    return jnp.where(invalid, trash, t)


def _tc1_body(x_ref, w1_ref, dpa_ref, dpb_ref, dst_ref,
              hsa_ref, hsb_ref, dinv_ref, dst0_ref, dst1_ref):
    dinv = lax.rsqrt(dpa_ref[...] + dpb_ref[...] + 1.0)
    h = jnp.dot(x_ref[...], w1_ref[...], preferred_element_type=_f32)
    hs = h * dinv
    hsa_ref[...] = hs[:, : D_HID // 2]
    hsb_ref[...] = hs[:, D_HID // 2:]
    dinv_ref[...] = dinv
    d = dst_ref[...]
    dst0_ref[...] = _dst_transform(d, 0)
    dst1_ref[...] = _dst_transform(d, 1)


def _tc1(x, W1, dpa, dpb, dst2d):
    grid = (N // BN,)
    return pl.pallas_call(
        _tc1_body,
        grid=grid,
        in_specs=[
            pl.BlockSpec((BN, D_IN), lambda i: (i, 0)),
            pl.BlockSpec((D_IN, D_HID), lambda i: (0, 0)),
            pl.BlockSpec((BN, 1), lambda i: (i, 0)),
            pl.BlockSpec((BN, 1), lambda i: (i, 0)),
            pl.BlockSpec((BE, K), lambda i: (i, 0)),
        ],
        out_specs=[
            pl.BlockSpec((BN, D_HID // 2), lambda i: (i, 0)),
            pl.BlockSpec((BN, D_HID // 2), lambda i: (i, 0)),
            pl.BlockSpec((BN, 1), lambda i: (i, 0)),
            pl.BlockSpec((BE, K), lambda i: (i, 0)),
            pl.BlockSpec((BE, K), lambda i: (i, 0)),
        ],
        out_shape=[
            jax.ShapeDtypeStruct((N, D_HID // 2), _f32),
            jax.ShapeDtypeStruct((N, D_HID // 2), _f32),
            jax.ShapeDtypeStruct((N, 1), _f32),
            jax.ShapeDtypeStruct((EC, K), jnp.int32),
            jax.ShapeDtypeStruct((EC, K), jnp.int32),
        ],
    )(x, W1, dpa, dpb, dst2d)


def _tc2_body(agga_ref, aggb_ref, dinv_ref, b1a_ref, b1b_ref,
              w2a_ref, w2b_ref, hs2_ref):
    dinv = dinv_ref[...]
    ta = jax.nn.relu(agga_ref[...] * dinv + b1a_ref[...])
    tb = jax.nn.relu(aggb_ref[...] * dinv + b1b_ref[...])
    h2 = (jnp.dot(ta, w2a_ref[...], preferred_element_type=_f32)
          + jnp.dot(tb, w2b_ref[...], preferred_element_type=_f32))
    hs2_ref[...] = h2 * dinv


def _tc2(agg_a, agg_b, dinv, b1a, b1b, W2a, W2b):
    grid = (N // BN,)
    half = D_HID // 2
    return pl.pallas_call(
        _tc2_body,
        grid=grid,
        in_specs=[
            pl.BlockSpec((BN, half), lambda i: (i, 0)),
            pl.BlockSpec((BN, half), lambda i: (i, 0)),
            pl.BlockSpec((BN, 1), lambda i: (i, 0)),
            pl.BlockSpec((1, half), lambda i: (0, 0)),
            pl.BlockSpec((1, half), lambda i: (0, 0)),
            pl.BlockSpec((half, D_OUT), lambda i: (0, 0)),
            pl.BlockSpec((half, D_OUT), lambda i: (0, 0)),
        ],
        out_specs=pl.BlockSpec((BN, D_OUT), lambda i: (i, 0)),
        out_shape=jax.ShapeDtypeStruct((N, D_OUT), _f32),
    )(agg_a, agg_b, dinv, b1a, b1b, W2a, W2b)


def _tc3_body(p_ref, dinv_ref, b2_ref, out_ref):
    out_ref[...] = p_ref[...] * dinv_ref[...] + b2_ref[...]


def _tc3(p, dinv, b2r):
    grid = (N // BN,)
    return pl.pallas_call(
        _tc3_body,
        grid=grid,
        in_specs=[
            pl.BlockSpec((BN, D_OUT), lambda i: (i, 0)),
            pl.BlockSpec((BN, 1), lambda i: (i, 0)),
            pl.BlockSpec((1, D_OUT), lambda i: (0, 0)),
        ],
        out_specs=pl.BlockSpec((BN, D_OUT), lambda i: (i, 0)),
        out_shape=jax.ShapeDtypeStruct((N, D_OUT), _f32),
    )(p, dinv, b2r)


def kernel(x, adj_t, W1, b1, W2, b2):
    assert x.shape == (N, D_IN) and adj_t.shape == (2, E)
    assert W1.shape == (D_IN, D_HID) and W2.shape == (D_HID, D_OUT)

    pad = EP - E
    src2d = jnp.concatenate(
        [adj_t[0], jnp.zeros((pad,), jnp.int32)]).reshape(EC, K)
    dst_p = jnp.concatenate(
        [adj_t[1], jnp.full((pad,), N, jnp.int32)]).reshape(EC, K)
    dum = jnp.zeros((K, 128), _f32)

    dega, degb = _sc_deg(dst_p)
    dpa = dega[:N].reshape(N, 1)
    dpb = degb[:N].reshape(N, 1)

    hs_a, hs_b, dinv, dst0, dst1 = _tc1(x, W1, dpa, dpb, dst_p)
    agg_a, agg_b = _sc_agg_dual(src2d, dst0, dst1, hs_a, hs_b, dum)

    b1a = b1[: D_HID // 2].reshape(1, -1)
    b1b = b1[D_HID // 2:].reshape(1, -1)
    hs2 = _tc2(agg_a, agg_b, dinv, b1a, b1b, W2[: D_HID // 2], W2[D_HID // 2:])

    p = _sc_agg(src2d, dst0, dst1, hs2, dum)
    return _tc3(p, dinv, b2.reshape(1, -1))
